# baseline (speedup 1.0000x reference)
import functools

import jax
import jax.numpy as jnp
from jax import lax
from jax.experimental import pallas as pl
from jax.experimental.pallas import tpu as pltpu
from jax.experimental.pallas import tpu_sc as plsc

F32 = jnp.float32
I32 = jnp.int32

_N_T = 10000
_N_S = 40000
_N_GRAPH = 512
_CHANNELS = [2, 2, 2, 2]
_FILTERS = [64, 128, 256, 512]
_RB = 128


def _round_up(x, m):
    return (x + m - 1) // m * m


def _mm_body(*refs, n_a, has_bias, n_c, relu):
    i = 0
    a_refs = refs[i:i + n_a]; i += n_a
    w_refs = refs[i:i + n_a]; i += n_a
    b_ref = refs[i] if has_bias else None
    i += 1 if has_bias else 0
    c_refs = refs[i:i + n_c]; i += n_c
    o_ref = refs[i]
    acc = jnp.dot(a_refs[0][...], w_refs[0][...], preferred_element_type=F32)
    for j in range(1, n_a):
        acc = acc + jnp.dot(a_refs[j][...], w_refs[j][...],
                            preferred_element_type=F32)
    if has_bias:
        acc = acc + b_ref[...]
    if n_c >= 1:
        acc = acc + c_refs[0][...]
    if n_c == 2:
        acc = acc - c_refs[1][...]
    if relu:
        acc = jnp.maximum(acc, 0.0)
    o_ref[...] = acc


def _mm(a_list, w_list, bias=None, c_list=(), relu=False):
    M = a_list[0].shape[0]
    N = w_list[0].shape[1]
    if M % 400 == 0:
        BM = 400
    elif M % 256 == 0:
        BM = 256
    else:
        BM = M
    grid = (M // BM,)
    in_specs = []
    args = []
    for a in a_list:
        K = a.shape[1]
        in_specs.append(pl.BlockSpec((BM, K), lambda i: (i, 0)))
        args.append(a)
    for w in w_list:
        K = w.shape[0]
        in_specs.append(pl.BlockSpec((K, N), lambda i: (0, 0)))
        args.append(w)
    has_bias = bias is not None
    if has_bias:
        in_specs.append(pl.BlockSpec((1, N), lambda i: (0, 0)))
        args.append(bias.reshape(1, N))
    for c in c_list:
        in_specs.append(pl.BlockSpec((BM, N), lambda i: (i, 0)))
        args.append(c)
    body = functools.partial(_mm_body, n_a=len(a_list), has_bias=has_bias,
                             n_c=len(c_list), relu=relu)
    return pl.pallas_call(
        body,
        grid=grid,
        in_specs=in_specs,
        out_specs=pl.BlockSpec((BM, N), lambda i: (i, 0)),
        out_shape=jax.ShapeDtypeStruct((M, N), F32),
    )(*args)


def _segmm_body(off_ref, rid_ref, val_ref, g_ref, o_ref, acc_ref, *, kw):
    i = pl.program_id(0)
    k = pl.program_id(1)

    @pl.when(k == 0)
    def _():
        acc_ref[...] = jnp.zeros_like(acc_ref)

    rid = rid_ref[0]
    val = val_ref[0]
    rows = lax.broadcasted_iota(I32, (_RB, 1), 0) + i * _RB
    S = jnp.where(rid == rows, val, 0.0)
    acc_ref[...] += jnp.dot(S, g_ref[...], preferred_element_type=F32)

    @pl.when(k == kw - 1)
    def _():
        o_ref[...] = acc_ref[...]


def _segmm(offs, rowid, val, g, rpad, kw):
    E, D = g.shape
    assert E % 128 == 0 and rpad % _RB == 0
    eb = E // 128
    rid3 = rowid.reshape(eb, 1, 128)
    val3 = val.reshape(eb, 1, 128)

    def win(i, k, off_ref):
        kb0 = jnp.minimum(off_ref[i * _RB] // 128, eb - kw)
        return kb0 + k

    grid_spec = pltpu.PrefetchScalarGridSpec(
        num_scalar_prefetch=1,
        grid=(rpad // _RB, kw),
        in_specs=[
            pl.BlockSpec((1, 1, 128), lambda i, k, off: (win(i, k, off), 0, 0)),
            pl.BlockSpec((1, 1, 128), lambda i, k, off: (win(i, k, off), 0, 0)),
            pl.BlockSpec((128, D), lambda i, k, off: (win(i, k, off), 0)),
        ],
        out_specs=pl.BlockSpec((_RB, D), lambda i, k, off: (i, 0)),
        scratch_shapes=[pltpu.VMEM((_RB, D), F32)],
    )
    return pl.pallas_call(
        functools.partial(_segmm_body, kw=kw),
        grid_spec=grid_spec,
        out_shape=jax.ShapeDtypeStruct((rpad, D), F32),
    )(offs, rid3, val3, g)


def _bn_stats_body(x_ref, o_ref):
    i = pl.program_id(0)

    @pl.when(i == 0)
    def _():
        o_ref[...] = jnp.zeros_like(o_ref)

    x = x_ref[...]
    o_ref[0:1, :] += jnp.sum(x, axis=0, keepdims=True)
    o_ref[1:2, :] += jnp.sum(x * x, axis=0, keepdims=True)


def _bn_apply_body(x_ref, st_ref, o_ref, *, m_count):
    x = x_ref[...]
    s = st_ref[0:1, :]
    s2 = st_ref[1:2, :]
    mu = s / m_count
    var = s2 / m_count - mu * mu
    y = (x - mu) * lax.rsqrt(var + 1e-5)
    o_ref[...] = jnp.maximum(y, 0.0)


def _bn_relu(x):
    M, N = x.shape
    BM = 400 if M % 400 == 0 else M
    stats = pl.pallas_call(
        _bn_stats_body,
        grid=(M // BM,),
        in_specs=[pl.BlockSpec((BM, N), lambda i: (i, 0))],
        out_specs=pl.BlockSpec((8, N), lambda i: (0, 0)),
        out_shape=jax.ShapeDtypeStruct((8, N), F32),
    )(x)
    return pl.pallas_call(
        functools.partial(_bn_apply_body, m_count=float(M)),
        grid=(M // BM,),
        in_specs=[
            pl.BlockSpec((BM, N), lambda i: (i, 0)),
            pl.BlockSpec((8, N), lambda i: (0, 0)),
        ],
        out_specs=pl.BlockSpec((BM, N), lambda i: (i, 0)),
        out_shape=jax.ShapeDtypeStruct((M, N), F32),
    )(x, stats)


def _sc_gather(x, idx):
    E = idx.shape[0]
    T, D = x.shape
    if D % 128 != 0:
        dp = _round_up(D, 128)
        out = _sc_gather(jnp.pad(x, ((0, 0), (0, dp - D))), idx)
        return out[:, :D]
    C = 128
    assert E % C == 0
    nch = E // C
    idx2 = idx.reshape(nch, C)
    mesh = plsc.VectorSubcoreMesh(core_axis_name="c", subcore_axis_name="s")

    @functools.partial(
        pl.kernel,
        mesh=mesh,
        out_type=jax.ShapeDtypeStruct((E, D), F32),
        scratch_types=[
            pltpu.VMEM((C,), I32),
            pltpu.VMEM((C, D), F32),
            pltpu.SemaphoreType.DMA,
        ],
    )
    def k(x_hbm, idx_hbm, out_hbm, idx_v, rows_v, sem):
        wid = lax.axis_index("s") * 2 + lax.axis_index("c")
        nj = (nch - wid + 31) // 32

        def body(j, carry):
            ci = wid + 32 * j
            pltpu.sync_copy(idx_hbm.at[ci], idx_v)
            pltpu.async_copy(x_hbm.at[idx_v], rows_v, sem).wait()
            pltpu.sync_copy(rows_v, out_hbm.at[pl.ds(ci * C, C)])
            return carry

        lax.fori_loop(0, nj, body, 0)

    return k(x, idx2)


def _csr_by_row(keys, cols, vals, rpad):
    k, c, v = lax.sort((keys, cols, vals), num_keys=1)
    offs = jnp.searchsorted(k, jnp.arange(rpad + 1, dtype=I32)).astype(I32)
    return offs, k.astype(I32), v, c.astype(I32)


def kernel(x_t, x_s, edge_index, edge_index_t, edge_weight_t, edge_index_s,
           edge_weight_s, n_batch, s_batch, params):
    nt, ns, ng = _N_T, _N_S, _N_GRAPH
    rt = _round_up(nt, _RB)
    rs = _round_up(ns, _RB)
    es = edge_index.shape[1]

    src = edge_index[0].astype(I32)
    dst = edge_index[1].astype(I32)
    ar = jnp.arange(es, dtype=I32)
    keysE = jnp.concatenate([src, dst])
    colsE = jnp.concatenate([ar, ar])
    sgnE = jnp.concatenate([jnp.full((es,), -1.0, F32),
                            jnp.full((es,), 1.0, F32)])
    offE, rowE, sgnE_s, colE = _csr_by_row(keysE, colsE, sgnE, rt)
    degE = jnp.maximum((offE[1:nt + 1] - offE[:nt]).astype(F32), 1.0)
    invdeg = 1.0 / degE
    valE = sgnE_s * jnp.take(invdeg, rowE, axis=0)

    offT, rowT, valT, colT = _csr_by_row(
        edge_index_t[1].astype(I32), edge_index_t[0].astype(I32),
        edge_weight_t, rt)
    offS, rowS, valS, colS = _csr_by_row(
        edge_index_s[1].astype(I32), edge_index_s[0].astype(I32),
        edge_weight_s, rs)

    nbat = n_batch.astype(I32)
    sbat = s_batch.astype(I32)
    offGn = jnp.searchsorted(nbat, jnp.arange(ng + 1, dtype=I32)).astype(I32)
    cGn = jnp.maximum((offGn[1:] - offGn[:-1]).astype(F32), 1.0)
    valGn = jnp.take(1.0 / cGn, nbat, axis=0)
    offGs = jnp.searchsorted(sbat, jnp.arange(ng + 1, dtype=I32)).astype(I32)
    cGs = jnp.maximum((offGs[1:] - offGs[:-1]).astype(F32), 1.0)
    valGs = jnp.take(1.0 / cGs, sbat, axis=0)
    ntp = _round_up(nt, 128)
    nsp = _round_up(ns, 128)
    rowGn = jnp.pad(nbat, (0, ntp - nt), constant_values=-1)
    valGn = jnp.pad(valGn, (0, ntp - nt))
    rowGs = jnp.pad(sbat, (0, nsp - ns), constant_values=-1)
    valGs = jnp.pad(valGs, (0, nsp - ns))

    n2e_idx = jnp.concatenate([dst, src])

    emb = params["emb"]

    def init_stage(x, lin):
        Wi, bi = lin[0][0], lin[1]
        it = jnp.clip(x[:, 0].astype(I32), 0, 27)
        oh = jax.nn.one_hot(it, 28, dtype=F32)
        feats = x[:, 1:]
        M = x.shape[0]
        A = jnp.concatenate(
            [oh, feats, jnp.zeros((M, 64 - 28 - feats.shape[1]), F32)], axis=1)
        Wtop = emb @ Wi[:57]
        W = jnp.concatenate(
            [Wtop, Wi[57:], jnp.zeros((64 - 28 - 7, 64), F32)], axis=0)
        y = _mm([A], [W], bias=bi)
        return _bn_relu(y)

    xt0 = init_stage(x_t, params["init_t"])
    xs0 = init_stage(x_s, params["init_s"])

    KW_T = 16
    KW_S = 8
    KW_E = 16
    KW_GN = 32
    KW_GS = 96

    def conv(x, offs, rowid, val, col, rpad, kw, Ws, b):
        W0, W1 = Ws
        g = _sc_gather(x, col)
        p = _segmm(offs, rowid, val, g, rpad, kw)
        p = p[:x.shape[0]]
        y = _mm([x, p], [W0 + W1, -W1], bias=b)
        return _bn_relu(y)

    bi = 0
    for i in range(len(_CHANNELS)):
        for _ in range(_CHANNELS[i]):
            blk = params["blocks"][bi]
            bi += 1
            d = xt0.shape[1]
            Wt, bt = blk["int_t"]
            Ws_, bs_ = blk["int_s"]
            Yt = _mm([xs0], [Wt[d:]])
            gE = _sc_gather(Yt, colE)
            e2n = _segmm(offE, rowE, valE, gE, rt, KW_E)[:nt]
            xt = _mm([xt0], [Wt[:d]], bias=bt, c_list=[e2n], relu=True)
            Zt = _mm([xt0], [Ws_[d:]])
            gN = _sc_gather(Zt, n2e_idx)
            xs = _mm([xs0], [Ws_[:d]], bias=bs_,
                     c_list=[gN[:es], gN[es:]], relu=True)
            xt = conv(xt, offT, rowT, valT, colT, rt, KW_T,
                      blk["conv_t"][0], blk["conv_t"][1])
            xs = conv(xs, offS, rowS, valS, colS, rs, KW_S,
                      blk["conv_s"][0], blk["conv_s"][1])
            xt0 = jnp.concatenate([xt0, xt], axis=1)
            xs0 = jnp.concatenate([xs0, xs], axis=1)

    xtp = jnp.pad(xt, ((0, ntp - nt), (0, 0)))
    xsp = jnp.pad(xs, ((0, nsp - ns), (0, 0)))
    pt = _segmm(offGn, rowGn, valGn, xtp, ng, KW_GN)
    ps = _segmm(offGs, rowGs, valGs, xsp, ng, KW_GS)
    pooled = jnp.concatenate([ps, pt], axis=1)
    Wo, bo = params["out"]
    Wo = jnp.pad(Wo, ((0, 0), (0, 127)))
    bo = jnp.pad(bo, (0, 127))
    out = _mm([pooled], [Wo], bias=bo)
    return out[:, :1]


# speedup vs baseline: 1.7853x; 1.7853x over previous
import functools

import jax
import jax.numpy as jnp
from jax import lax
from jax.experimental import pallas as pl
from jax.experimental.pallas import tpu as pltpu
from jax.experimental.pallas import tpu_sc as plsc

F32 = jnp.float32
I32 = jnp.int32

_N_T = 10000
_N_S = 40000
_N_GRAPH = 512
_CHANNELS = [2, 2, 2, 2]
_FILTERS = [64, 128, 256, 512]
_RB = 128


def _round_up(x, m):
    return (x + m - 1) // m * m


def _mm_body(*refs, n_a, has_bias, n_c, relu):
    i = 0
    a_refs = refs[i:i + n_a]; i += n_a
    w_refs = refs[i:i + n_a]; i += n_a
    b_ref = refs[i] if has_bias else None
    i += 1 if has_bias else 0
    c_refs = refs[i:i + n_c]; i += n_c
    o_ref = refs[i]
    acc = jnp.dot(a_refs[0][...], w_refs[0][...], preferred_element_type=F32)
    for j in range(1, n_a):
        acc = acc + jnp.dot(a_refs[j][...], w_refs[j][...],
                            preferred_element_type=F32)
    if has_bias:
        acc = acc + b_ref[...]
    if n_c >= 1:
        acc = acc + c_refs[0][...]
    if n_c == 2:
        acc = acc - c_refs[1][...]
    if relu:
        acc = jnp.maximum(acc, 0.0)
    o_ref[...] = acc


def _mm(a_list, w_list, bias=None, c_list=(), relu=False):
    M = a_list[0].shape[0]
    N = w_list[0].shape[1]
    if M % 400 == 0:
        BM = 400
    elif M % 256 == 0:
        BM = 256
    else:
        BM = M
    grid = (M // BM,)
    in_specs = []
    args = []
    for a in a_list:
        K = a.shape[1]
        in_specs.append(pl.BlockSpec((BM, K), lambda i: (i, 0)))
        args.append(a)
    for w in w_list:
        K = w.shape[0]
        in_specs.append(pl.BlockSpec((K, N), lambda i: (0, 0)))
        args.append(w)
    has_bias = bias is not None
    if has_bias:
        in_specs.append(pl.BlockSpec((1, N), lambda i: (0, 0)))
        args.append(bias.reshape(1, N))
    for c in c_list:
        in_specs.append(pl.BlockSpec((BM, N), lambda i: (i, 0)))
        args.append(c)
    body = functools.partial(_mm_body, n_a=len(a_list), has_bias=has_bias,
                             n_c=len(c_list), relu=relu)
    return pl.pallas_call(
        body,
        grid=grid,
        in_specs=in_specs,
        out_specs=pl.BlockSpec((BM, N), lambda i: (i, 0)),
        out_shape=jax.ShapeDtypeStruct((M, N), F32),
    )(*args)


def _segmm_body(off_ref, rid1_ref, val1_ref, g1_ref, rid2_ref, val2_ref,
                g2_ref, o_ref, *, kw, rb, lastb):
    i = pl.program_id(0)
    rows = lax.broadcasted_iota(I32, (rb, 1), 0) + i * rb
    a = off_ref[i * rb] // (kw * 128)
    dup = a >= lastb
    acc = jnp.zeros_like(o_ref)
    for j in range(kw):
        S = jnp.where(rid1_ref[j] == rows, val1_ref[j], 0.0)
        acc = acc + jnp.dot(S, g1_ref[j * 128:(j + 1) * 128, :],
                            preferred_element_type=F32)
    acc2 = jnp.zeros_like(o_ref)
    for j in range(kw):
        S = jnp.where(rid2_ref[j] == rows, val2_ref[j], 0.0)
        acc2 = acc2 + jnp.dot(S, g2_ref[j * 128:(j + 1) * 128, :],
                              preferred_element_type=F32)
    o_ref[...] = acc + jnp.where(dup, 0.0, 1.0) * acc2


def _segmm(offs, rowid, val, g, rpad, kw, rb=_RB):
    E, D = g.shape
    W = kw * 128
    assert E % W == 0 and rpad % rb == 0
    eb = E // 128
    lastb = E // W - 1
    rid3 = rowid.reshape(eb, 1, 128)
    val3 = val.reshape(eb, 1, 128)

    def win1(i, off_ref):
        return jnp.minimum(off_ref[i * rb] // W, lastb)

    def win2(i, off_ref):
        return jnp.minimum(off_ref[i * rb] // W + 1, lastb)

    grid_spec = pltpu.PrefetchScalarGridSpec(
        num_scalar_prefetch=1,
        grid=(rpad // rb,),
        in_specs=[
            pl.BlockSpec((kw, 1, 128), lambda i, off: (win1(i, off), 0, 0)),
            pl.BlockSpec((kw, 1, 128), lambda i, off: (win1(i, off), 0, 0)),
            pl.BlockSpec((W, D), lambda i, off: (win1(i, off), 0)),
            pl.BlockSpec((kw, 1, 128), lambda i, off: (win2(i, off), 0, 0)),
            pl.BlockSpec((kw, 1, 128), lambda i, off: (win2(i, off), 0, 0)),
            pl.BlockSpec((W, D), lambda i, off: (win2(i, off), 0)),
        ],
        out_specs=pl.BlockSpec((rb, D), lambda i, off: (i, 0)),
    )
    return pl.pallas_call(
        functools.partial(_segmm_body, kw=kw, rb=rb, lastb=lastb),
        grid_spec=grid_spec,
        out_shape=jax.ShapeDtypeStruct((rpad, D), F32),
    )(offs, rid3, val3, g, rid3, val3, g)


def _bn_stats_body(x_ref, o_ref):
    i = pl.program_id(0)

    @pl.when(i == 0)
    def _():
        o_ref[...] = jnp.zeros_like(o_ref)

    x = x_ref[...]
    o_ref[0:1, :] += jnp.sum(x, axis=0, keepdims=True)
    o_ref[1:2, :] += jnp.sum(x * x, axis=0, keepdims=True)


def _bn_apply_body(x_ref, st_ref, o_ref, *, m_count):
    x = x_ref[...]
    s = st_ref[0:1, :]
    s2 = st_ref[1:2, :]
    mu = s / m_count
    var = s2 / m_count - mu * mu
    y = (x - mu) * lax.rsqrt(var + 1e-5)
    o_ref[...] = jnp.maximum(y, 0.0)


def _bn_relu(x):
    M, N = x.shape
    BM = 400 if M % 400 == 0 else M
    stats = pl.pallas_call(
        _bn_stats_body,
        grid=(M // BM,),
        in_specs=[pl.BlockSpec((BM, N), lambda i: (i, 0))],
        out_specs=pl.BlockSpec((8, N), lambda i: (0, 0)),
        out_shape=jax.ShapeDtypeStruct((8, N), F32),
    )(x)
    return pl.pallas_call(
        functools.partial(_bn_apply_body, m_count=float(M)),
        grid=(M // BM,),
        in_specs=[
            pl.BlockSpec((BM, N), lambda i: (i, 0)),
            pl.BlockSpec((8, N), lambda i: (0, 0)),
        ],
        out_specs=pl.BlockSpec((BM, N), lambda i: (i, 0)),
        out_shape=jax.ShapeDtypeStruct((M, N), F32),
    )(x, stats)


def _sc_gather(x, idx):
    E = idx.shape[0]
    T, D = x.shape
    if D % 128 != 0:
        dp = _round_up(D, 128)
        out = _sc_gather(jnp.pad(x, ((0, 0), (0, dp - D))), idx)
        return out[:, :D]
    C = 128
    assert E % C == 0
    nch = E // C
    idx2 = idx.reshape(nch, C)
    mesh = plsc.VectorSubcoreMesh(core_axis_name="c", subcore_axis_name="s")

    @functools.partial(
        pl.kernel,
        mesh=mesh,
        out_type=jax.ShapeDtypeStruct((E, D), F32),
        scratch_types=[
            pltpu.VMEM((C,), I32),
            pltpu.VMEM((C, D), F32),
            pltpu.SemaphoreType.DMA,
        ],
    )
    def k(x_hbm, idx_hbm, out_hbm, idx_v, rows_v, sem):
        wid = lax.axis_index("s") * 2 + lax.axis_index("c")
        nj = (nch - wid + 31) // 32

        def body(j, carry):
            ci = wid + 32 * j
            pltpu.sync_copy(idx_hbm.at[ci], idx_v)
            pltpu.async_copy(x_hbm.at[idx_v], rows_v, sem).wait()
            pltpu.sync_copy(rows_v, out_hbm.at[pl.ds(ci * C, C)])
            return carry

        lax.fori_loop(0, nj, body, 0)

    return k(x, idx2)


def _csr_by_row(keys, cols, vals, rpad, epad):
    k, c, v = lax.sort((keys, cols, vals), num_keys=1)
    offs = jnp.searchsorted(k, jnp.arange(rpad + 1, dtype=I32)).astype(I32)
    pad = epad - keys.shape[0]
    k = jnp.pad(k.astype(I32), (0, pad), constant_values=-1)
    v = jnp.pad(v, (0, pad))
    c = jnp.pad(c.astype(I32), (0, pad))
    return offs, k, v, c


def kernel(x_t, x_s, edge_index, edge_index_t, edge_weight_t, edge_index_s,
           edge_weight_s, n_batch, s_batch, params):
    nt, ns, ng = _N_T, _N_S, _N_GRAPH
    rt = _round_up(nt, _RB)
    rs = _round_up(ns, _RB)
    es = edge_index.shape[1]

    src = edge_index[0].astype(I32)
    dst = edge_index[1].astype(I32)
    ar = jnp.arange(es, dtype=I32)
    keysE = jnp.concatenate([src, dst])
    colsE = jnp.concatenate([ar, ar])
    sgnE = jnp.concatenate([jnp.full((es,), -1.0, F32),
                            jnp.full((es,), 1.0, F32)])
    KW_T = 16
    KW_S = 8
    KW_E = 16
    epT = _round_up(2 * es, KW_E * 128)
    epS = _round_up(edge_index_s.shape[1], KW_S * 128)
    offE, rowE, sgnE_s, colE = _csr_by_row(keysE, colsE, sgnE, rt, epT)
    degE = jnp.maximum((offE[1:nt + 1] - offE[:nt]).astype(F32), 1.0)
    invdeg = 1.0 / degE
    valE = sgnE_s * jnp.take(invdeg, jnp.clip(rowE, 0, nt - 1), axis=0)
    valE = jnp.where(rowE >= 0, valE, 0.0)

    offT, rowT, valT, colT = _csr_by_row(
        edge_index_t[1].astype(I32), edge_index_t[0].astype(I32),
        edge_weight_t, rt, epT)
    offS, rowS, valS, colS = _csr_by_row(
        edge_index_s[1].astype(I32), edge_index_s[0].astype(I32),
        edge_weight_s, rs, epS)

    nbat = n_batch.astype(I32)
    sbat = s_batch.astype(I32)
    offGn = jnp.searchsorted(nbat, jnp.arange(ng + 1, dtype=I32)).astype(I32)
    cGn = jnp.maximum((offGn[1:] - offGn[:-1]).astype(F32), 1.0)
    valGn = jnp.take(1.0 / cGn, nbat, axis=0)
    offGs = jnp.searchsorted(sbat, jnp.arange(ng + 1, dtype=I32)).astype(I32)
    cGs = jnp.maximum((offGs[1:] - offGs[:-1]).astype(F32), 1.0)
    valGs = jnp.take(1.0 / cGs, sbat, axis=0)
    KW_GN = 8
    KW_GS = 24
    ntp = _round_up(nt, KW_GN * 128)
    nsp = _round_up(ns, KW_GS * 128)
    rowGn = jnp.pad(nbat, (0, ntp - nt), constant_values=-1)
    valGn = jnp.pad(valGn, (0, ntp - nt))
    rowGs = jnp.pad(sbat, (0, nsp - ns), constant_values=-1)
    valGs = jnp.pad(valGs, (0, nsp - ns))

    n2e_idx = jnp.concatenate([dst, src])

    emb = params["emb"]

    def init_stage(x, lin):
        Wi, bi = lin[0][0], lin[1]
        it = jnp.clip(x[:, 0].astype(I32), 0, 27)
        oh = jax.nn.one_hot(it, 28, dtype=F32)
        feats = x[:, 1:]
        M = x.shape[0]
        A = jnp.concatenate(
            [oh, feats, jnp.zeros((M, 64 - 28 - feats.shape[1]), F32)], axis=1)
        Wtop = emb @ Wi[:57]
        W = jnp.concatenate(
            [Wtop, Wi[57:], jnp.zeros((64 - 28 - 7, 64), F32)], axis=0)
        y = _mm([A], [W], bias=bi)
        return _bn_relu(y)

    xt0 = init_stage(x_t, params["init_t"])
    xs0 = init_stage(x_s, params["init_s"])

    def conv(x, offs, rowid, val, col, rpad, kw, Ws, b):
        W0, W1 = Ws
        g = _sc_gather(x, col)
        p = _segmm(offs, rowid, val, g, rpad, kw)
        p = p[:x.shape[0]]
        y = _mm([x, p], [W0 + W1, -W1], bias=b)
        return _bn_relu(y)

    bi = 0
    for i in range(len(_CHANNELS)):
        for _ in range(_CHANNELS[i]):
            blk = params["blocks"][bi]
            bi += 1
            d = xt0.shape[1]
            Wt, bt = blk["int_t"]
            Ws_, bs_ = blk["int_s"]
            Yt = _mm([xs0], [Wt[d:]])
            gE = _sc_gather(Yt, colE)
            e2n = _segmm(offE, rowE, valE, gE, rt, KW_E)[:nt]
            xt = _mm([xt0], [Wt[:d]], bias=bt, c_list=[e2n], relu=True)
            Zt = _mm([xt0], [Ws_[d:]])
            gN = _sc_gather(Zt, n2e_idx)
            xs = _mm([xs0], [Ws_[:d]], bias=bs_,
                     c_list=[gN[:es], gN[es:]], relu=True)
            xt = conv(xt, offT, rowT, valT, colT, rt, KW_T,
                      blk["conv_t"][0], blk["conv_t"][1])
            xs = conv(xs, offS, rowS, valS, colS, rs, KW_S,
                      blk["conv_s"][0], blk["conv_s"][1])
            xt0 = jnp.concatenate([xt0, xt], axis=1)
            xs0 = jnp.concatenate([xs0, xs], axis=1)

    xtp = jnp.pad(xt, ((0, ntp - nt), (0, 0)))
    xsp = jnp.pad(xs, ((0, nsp - ns), (0, 0)))
    pt = _segmm(offGn, rowGn, valGn, xtp, ng, KW_GN, rb=32)
    ps = _segmm(offGs, rowGs, valGs, xsp, ng, KW_GS, rb=32)
    pooled = jnp.concatenate([ps, pt], axis=1)
    Wo, bo = params["out"]
    Wo = jnp.pad(Wo, ((0, 0), (0, 127)))
    bo = jnp.pad(bo, (0, 127))
    out = _mm([pooled], [Wo], bias=bo)
    return out[:, :1]
